# Initial kernel scaffold; baseline (speedup 1.0000x reference)
import jax
import jax.numpy as jnp
from jax.experimental import pallas as pl


def kernel(atom_num, nbr_idx, nbr_fea, crystal_atom_idx, uni_idx, uni_count, emb, Wf0, bf0, g1_0, b1_0, g2_0, b2_0, Wf1, bf1, g1_1, b1_1, g2_1, b2_1, Wf2, bf2, g1_2, b1_2, g2_2, b2_2, Wfc, bfc):
    raise NotImplementedError("write your pallas kernel here")


# speedup vs baseline: 3.0182x; 3.0182x over previous
import functools

import jax
import jax.numpy as jnp
from jax import lax
from jax.experimental import pallas as pl
from jax.experimental.pallas import tpu as pltpu
from jax.experimental.pallas import tpu_sc as plsc

AFL = 128
NFL = 16
MGL = 512
HID = 128
N_ATOMS = 10000
M_NBR = 32
BATCH = 16
APC = N_ATOMS // BATCH
EPS = 1e-5
N_EDGE = N_ATOMS * M_NBR

_NC, _NS = 2, 16
_NW = _NC * _NS


def _sc_gather_body(b_per_w, chunk, nbuf, table_hbm, idx_hbm, out_hbm,
                    idx_v, rows_v, sems):
    wid = lax.axis_index("s") * _NC + lax.axis_index("c")
    base = wid * b_per_w
    nch = b_per_w // chunk

    def start(slot, ch):
        off = base + ch * chunk
        pltpu.sync_copy(idx_hbm.at[pl.ds(off, chunk)], idx_v.at[slot])
        pltpu.make_async_copy(table_hbm.at[idx_v.at[slot]], rows_v.at[slot],
                              sems.at[slot]).start()

    def drain(slot, ch):
        off = base + ch * chunk
        pltpu.make_async_copy(table_hbm.at[idx_v.at[slot]], rows_v.at[slot],
                              sems.at[slot]).wait()
        pltpu.sync_copy(rows_v.at[slot], out_hbm.at[pl.ds(off, chunk)])

    if nch <= nbuf:
        for ch in range(nch):
            start(ch, ch)
        for ch in range(nch):
            drain(ch, ch)
        return

    assert nch % nbuf == 0
    ngrp = nch // nbuf
    for s in range(nbuf):
        start(s, s)

    def step(grp, carry):
        for s in range(nbuf):
            ch = grp * nbuf + s
            drain(s, ch)
            start(s, ch + nbuf)
        return carry

    lax.fori_loop(0, ngrp - 1, step, 0)
    for s in range(nbuf):
        drain(s, (ngrp - 1) * nbuf + s)


def _sc_gather(table, idx, chunk, nbuf=2):
    btot = idx.shape[0]
    d = table.shape[1]
    assert btot % _NW == 0
    b_per_w = btot // _NW
    assert b_per_w % chunk == 0 and chunk % 8 == 0 and chunk <= 128
    mesh = plsc.VectorSubcoreMesh(core_axis_name="c", subcore_axis_name="s")
    body = functools.partial(_sc_gather_body, b_per_w, chunk, nbuf)
    k = pl.kernel(
        body,
        out_type=jax.ShapeDtypeStruct((btot, d), table.dtype),
        mesh=mesh,
        scratch_types=[
            pltpu.VMEM((nbuf, chunk), jnp.int32),
            pltpu.VMEM((nbuf, chunk, d), table.dtype),
            pltpu.SemaphoreType.DMA((nbuf,)),
        ],
    )
    return k(table, idx)


def _prep_body(f_ref, ws_ref, bf_ref, a_ref):
    a_ref[...] = (
        jnp.dot(f_ref[...], ws_ref[...], preferred_element_type=jnp.float32)
        + bf_ref[...]
    )


def _edge_x(g_ref, nf_ref, a_ref, wn_ref, wq_ref, ba):
    x = jnp.dot(g_ref[...], wn_ref[...], preferred_element_type=jnp.float32)
    x = x + jnp.dot(nf_ref[...], wq_ref[...], preferred_element_type=jnp.float32)
    a3 = jnp.broadcast_to(a_ref[...][:, None, :], (ba, M_NBR, 2 * AFL))
    return x + a3.reshape(ba * M_NBR, 2 * AFL)


def _p1_body(ba, g_ref, nf_ref, a_ref, wn_ref, wq_ref, acc_ref):
    x = _edge_x(g_ref, nf_ref, a_ref, wn_ref, wq_ref, ba)
    s1 = jnp.sum(x, axis=0, keepdims=True)
    s2 = jnp.sum(x * x, axis=0, keepdims=True)

    @pl.when(pl.program_id(0) == 0)
    def _():
        acc_ref[...] = jnp.zeros_like(acc_ref)

    acc_ref[0:2, :] += jnp.concatenate([s1, s2], axis=0)


def _softplus(z):
    return jnp.maximum(z, 0.0) + jnp.log(1.0 + jnp.exp(-jnp.abs(z)))


def _p2_body(ba, g_ref, nf_ref, a_ref, wn_ref, wq_ref, st_ref, g1_ref, b1_ref,
             ns_ref, acc2_ref):
    cnt = float(N_EDGE)
    mu = st_ref[0:1, :] / cnt
    var = st_ref[1:2, :] / cnt - mu * mu
    s = g1_ref[...] * lax.rsqrt(var + EPS)
    t = b1_ref[...] - mu * s
    x = _edge_x(g_ref, nf_ref, a_ref, wn_ref, wq_ref, ba)
    y = x * s + t
    filt = 1.0 / (1.0 + jnp.exp(-y[:, :AFL]))
    core = _softplus(y[:, AFL:])
    prod = filt * core
    ns = jnp.sum(prod.reshape(ba, M_NBR, AFL), axis=1)
    ns_ref[...] = ns
    s1 = jnp.sum(ns, axis=0, keepdims=True)
    s2 = jnp.sum(ns * ns, axis=0, keepdims=True)

    @pl.when(pl.program_id(0) == 0)
    def _():
        acc2_ref[...] = jnp.zeros_like(acc2_ref)

    acc2_ref[0:2, :] += jnp.concatenate([s1, s2], axis=0)


def _bn2_scale(st_ref, g2_ref, b2_ref):
    cnt = float(N_ATOMS)
    mu = st_ref[0:1, :] / cnt
    var = st_ref[1:2, :] / cnt - mu * mu
    s = g2_ref[...] * lax.rsqrt(var + EPS)
    t = b2_ref[...] - mu * s
    return s, t


def _p3_body(ns_ref, f_ref, st_ref, g2_ref, b2_ref, wsn_ref, bfn_ref,
             newf_ref, an_ref):
    s, t = _bn2_scale(st_ref, g2_ref, b2_ref)
    nf = _softplus(f_ref[...] + ns_ref[...] * s + t)
    newf_ref[...] = nf
    an_ref[...] = (
        jnp.dot(nf, wsn_ref[...], preferred_element_type=jnp.float32)
        + bfn_ref[...]
    )


def _p3_final_body(ns_ref, f_ref, st_ref, g2_ref, b2_ref, wfc_ref, bfc_ref,
                   out_ref):
    s, t = _bn2_scale(st_ref, g2_ref, b2_ref)
    nf = _softplus(f_ref[...] + ns_ref[...] * s + t)
    out_ref[...] = (
        jnp.dot(nf, wfc_ref[...], preferred_element_type=jnp.float32)
        + bfc_ref[...]
    )


def _mask_body(x_ref, m_ref):
    m_ref[...] = (jnp.sum(x_ref[...], axis=1) != 0).astype(jnp.float32)


_SEQ = pltpu.CompilerParams(dimension_semantics=("arbitrary",))


def _full(shape):
    return pl.BlockSpec(shape, lambda i: (0,) * len(shape))


def _prep(f, ws, bf, bn=1000):
    grid = (N_ATOMS // bn,)
    return pl.pallas_call(
        _prep_body,
        grid=grid,
        in_specs=[
            pl.BlockSpec((bn, AFL), lambda i: (i, 0)),
            _full((AFL, 2 * AFL)),
            _full((1, 2 * AFL)),
        ],
        out_specs=pl.BlockSpec((bn, 2 * AFL), lambda i: (i, 0)),
        out_shape=jax.ShapeDtypeStruct((N_ATOMS, 2 * AFL), jnp.float32),
        compiler_params=_SEQ,
    )(f, ws, bf)


def _pass1(g, nf, a, wn, wq, ba=200):
    grid = (N_ATOMS // ba,)
    return pl.pallas_call(
        functools.partial(_p1_body, ba),
        grid=grid,
        in_specs=[
            pl.BlockSpec((ba * M_NBR, AFL), lambda i: (i, 0)),
            pl.BlockSpec((ba * M_NBR, NFL), lambda i: (i, 0)),
            pl.BlockSpec((ba, 2 * AFL), lambda i: (i, 0)),
            _full((AFL, 2 * AFL)),
            _full((NFL, 2 * AFL)),
        ],
        out_specs=_full((8, 2 * AFL)),
        out_shape=jax.ShapeDtypeStruct((8, 2 * AFL), jnp.float32),
        compiler_params=_SEQ,
    )(g, nf, a, wn, wq)


def _pass2(g, nf, a, wn, wq, st, g1, b1, ba=200):
    grid = (N_ATOMS // ba,)
    return pl.pallas_call(
        functools.partial(_p2_body, ba),
        grid=grid,
        in_specs=[
            pl.BlockSpec((ba * M_NBR, AFL), lambda i: (i, 0)),
            pl.BlockSpec((ba * M_NBR, NFL), lambda i: (i, 0)),
            pl.BlockSpec((ba, 2 * AFL), lambda i: (i, 0)),
            _full((AFL, 2 * AFL)),
            _full((NFL, 2 * AFL)),
            _full((8, 2 * AFL)),
            _full((1, 2 * AFL)),
            _full((1, 2 * AFL)),
        ],
        out_specs=[
            pl.BlockSpec((ba, AFL), lambda i: (i, 0)),
            _full((8, AFL)),
        ],
        out_shape=[
            jax.ShapeDtypeStruct((N_ATOMS, AFL), jnp.float32),
            jax.ShapeDtypeStruct((8, AFL), jnp.float32),
        ],
        compiler_params=_SEQ,
    )(g, nf, a, wn, wq, st, g1, b1)


def _pass3(ns, f, st2, g2, b2, wsn, bfn, bn=1000):
    grid = (N_ATOMS // bn,)
    return pl.pallas_call(
        _p3_body,
        grid=grid,
        in_specs=[
            pl.BlockSpec((bn, AFL), lambda i: (i, 0)),
            pl.BlockSpec((bn, AFL), lambda i: (i, 0)),
            _full((8, AFL)),
            _full((1, AFL)),
            _full((1, AFL)),
            _full((AFL, 2 * AFL)),
            _full((1, 2 * AFL)),
        ],
        out_specs=[
            pl.BlockSpec((bn, AFL), lambda i: (i, 0)),
            pl.BlockSpec((bn, 2 * AFL), lambda i: (i, 0)),
        ],
        out_shape=[
            jax.ShapeDtypeStruct((N_ATOMS, AFL), jnp.float32),
            jax.ShapeDtypeStruct((N_ATOMS, 2 * AFL), jnp.float32),
        ],
        compiler_params=_SEQ,
    )(ns, f, st2, g2, b2, wsn, bfn)


def _pass3_final(ns, f, st2, g2, b2, wfc, bfc, bn=1000):
    grid = (N_ATOMS // bn,)
    return pl.pallas_call(
        _p3_final_body,
        grid=grid,
        in_specs=[
            pl.BlockSpec((bn, AFL), lambda i: (i, 0)),
            pl.BlockSpec((bn, AFL), lambda i: (i, 0)),
            _full((8, AFL)),
            _full((1, AFL)),
            _full((1, AFL)),
            _full((AFL, HID)),
            _full((1, HID)),
        ],
        out_specs=pl.BlockSpec((bn, HID), lambda i: (i, 0)),
        out_shape=jax.ShapeDtypeStruct((N_ATOMS, HID), jnp.float32),
        compiler_params=_SEQ,
    )(ns, f, st2, g2, b2, wfc, bfc)


def _mask_kernel(rows, bn=1024):
    tot = rows.shape[0]
    grid = (tot // bn,)
    return pl.pallas_call(
        _mask_body,
        grid=grid,
        in_specs=[pl.BlockSpec((bn, HID), lambda i: (i, 0))],
        out_specs=pl.BlockSpec((bn,), lambda i: (i,)),
        out_shape=jax.ShapeDtypeStruct((tot,), jnp.float32),
        compiler_params=_SEQ,
    )(rows)


def kernel(atom_num, nbr_idx, nbr_fea, crystal_atom_idx, uni_idx, uni_count,
           emb, Wf0, bf0, g1_0, b1_0, g2_0, b2_0,
           Wf1, bf1, g1_1, b1_1, g2_1, b2_1,
           Wf2, bf2, g1_2, b1_2, g2_2, b2_2, Wfc, bfc):
    f32 = jnp.float32
    row = lambda v: v.reshape(1, -1).astype(f32)

    convs = []
    for Wf, bf, g1, b1, g2, b2 in (
        (Wf0, bf0, g1_0, b1_0, g2_0, b2_0),
        (Wf1, bf1, g1_1, b1_1, g2_1, b2_1),
        (Wf2, bf2, g1_2, b1_2, g2_2, b2_2),
    ):
        Wf = Wf.astype(f32)
        convs.append(dict(
            ws=Wf[:AFL], wn=Wf[AFL:2 * AFL], wq=Wf[2 * AFL:],
            bf=row(bf), g1=row(g1), b1=row(b1), g2=row(g2), b2=row(b2),
        ))

    npad = 10240
    an_pad = jnp.pad(atom_num.astype(jnp.int32), (0, npad - N_ATOMS))
    f = _sc_gather(emb.astype(f32), an_pad, chunk=80, nbuf=4)[:N_ATOMS]

    nbr_flat = nbr_idx.astype(jnp.int32).reshape(N_EDGE)
    nf_flat = nbr_fea.astype(f32).reshape(N_EDGE, NFL)

    a = _prep(f, convs[0]["ws"], convs[0]["bf"])
    for li, cv in enumerate(convs):
        g = _sc_gather(f, nbr_flat, chunk=80, nbuf=5)
        st = _pass1(g, nf_flat, a, cv["wn"], cv["wq"])
        ns, st2 = _pass2(g, nf_flat, a, cv["wn"], cv["wq"], st,
                         cv["g1"], cv["b1"])
        if li < 2:
            nxt = convs[li + 1]
            f, a = _pass3(ns, f, st2, cv["g2"], cv["b2"],
                          nxt["ws"], nxt["bf"])
        else:
            out_fc = _pass3_final(ns, f, st2, cv["g2"], cv["b2"],
                                  Wfc.astype(f32), row(bfc))

    flat_idx = jnp.take_along_axis(
        crystal_atom_idx.astype(jnp.int32), uni_idx.astype(jnp.int32), axis=1
    ).reshape(BATCH * MGL)
    rows = _sc_gather(out_fc, flat_idx, chunk=128, nbuf=2)
    maskf = _mask_kernel(rows)

    new_atom_fea = rows.reshape(BATCH, MGL, HID)
    mask = maskf.reshape(BATCH, MGL)
    hbond_label = jnp.full((BATCH, MGL), -100.0, dtype=f32)
    return new_atom_fea, mask, hbond_label, uni_idx
